# baseline (device time: 88723 ns/iter reference)
import jax
import jax.numpy as jnp
from jax import lax
from jax.experimental import pallas as pl
from jax.experimental.pallas import tpu as pltpu

N_DEV = 8
M = 1024
CH = M // N_DEV


def kernel(x, W1, W2):
    def body(x_ref, w1_ref, w2_ref, out_ref,
             acc_ref, send_buf, recv_buf, send_sems, recv_sems):
        my = lax.axis_index("i")
        right = lax.rem(my + 1, N_DEV)
        left = lax.rem(my + N_DEV - 1, N_DEV)

        barrier_sem = pltpu.get_barrier_semaphore()
        for nbr in (left, right):
            pl.semaphore_signal(
                barrier_sem, inc=1,
                device_id=(nbr,), device_id_type=pl.DeviceIdType.MESH,
            )
        pl.semaphore_wait(barrier_sem, 2)

        xb = x_ref[...].astype(jnp.bfloat16)
        w1 = w1_ref[...].astype(jnp.bfloat16)
        h = jnp.dot(xb, w1, preferred_element_type=jnp.float32)
        h = jnp.maximum(h, 0.0).astype(jnp.bfloat16)
        w2 = w2_ref[...].astype(jnp.bfloat16)
        acc_ref[...] = jnp.dot(h, w2, preferred_element_type=jnp.float32)

        for s in range(N_DEV - 1):
            send_idx = lax.rem(my - s + N_DEV, N_DEV)
            send_buf[...] = acc_ref[pl.ds(send_idx * CH, CH), :].astype(
                jnp.bfloat16)
            rdma = pltpu.make_async_remote_copy(
                src_ref=send_buf,
                dst_ref=recv_buf.at[s],
                send_sem=send_sems.at[s],
                recv_sem=recv_sems.at[s],
                device_id=(right,),
                device_id_type=pl.DeviceIdType.MESH,
            )
            rdma.start()
            rdma.wait()
            recv_idx = lax.rem(my - s - 1 + N_DEV, N_DEV)
            sl = pl.ds(recv_idx * CH, CH)
            acc_ref[sl, :] = acc_ref[sl, :] + recv_buf[s].astype(jnp.float32)

        red = lax.rem(my + 1, N_DEV)
        out_ref[pl.ds(red * CH, CH), :] = acc_ref[pl.ds(red * CH, CH), :]

        for t in range(N_DEV - 1):
            if t == 0:
                send_buf[...] = acc_ref[pl.ds(red * CH, CH), :].astype(
                    jnp.bfloat16)
                src = send_buf
            else:
                src = recv_buf.at[N_DEV - 1 + t - 1]
            rdma = pltpu.make_async_remote_copy(
                src_ref=src,
                dst_ref=recv_buf.at[N_DEV - 1 + t],
                send_sem=send_sems.at[N_DEV - 1 + t],
                recv_sem=recv_sems.at[N_DEV - 1 + t],
                device_id=(right,),
                device_id_type=pl.DeviceIdType.MESH,
            )
            rdma.start()
            rdma.wait()
            got = lax.rem(my - t + N_DEV, N_DEV)
            out_ref[pl.ds(got * CH, CH), :] = recv_buf[
                N_DEV - 1 + t].astype(jnp.float32)

    n_sems = 2 * (N_DEV - 1)
    return pl.pallas_call(
        body,
        out_shape=jax.ShapeDtypeStruct((M, M), jnp.float32),
        in_specs=[
            pl.BlockSpec(memory_space=pltpu.VMEM),
            pl.BlockSpec(memory_space=pltpu.VMEM),
            pl.BlockSpec(memory_space=pltpu.VMEM),
        ],
        out_specs=pl.BlockSpec(memory_space=pltpu.VMEM),
        scratch_shapes=[
            pltpu.VMEM((M, M), jnp.float32),
            pltpu.VMEM((CH, M), jnp.bfloat16),
            pltpu.VMEM((n_sems, CH, M), jnp.bfloat16),
            pltpu.SemaphoreType.DMA((n_sems,)),
            pltpu.SemaphoreType.DMA((n_sems,)),
        ],
        compiler_params=pltpu.CompilerParams(collective_id=0),
    )(x, W1, W2)


# device time: 50285 ns/iter; 1.7644x vs baseline; 1.7644x over previous
import jax
import jax.numpy as jnp
from jax import lax
from jax.experimental import pallas as pl
from jax.experimental.pallas import tpu as pltpu

N_DEV = 8
M = 1024
CH = M // N_DEV


def kernel(x, W1, W2):
    def body(x_ref, w1_ref, w2_ref, out_ref,
             own_ref, rs_send, rs_recv, ag_send, ag_recv,
             rs_send_sems, rs_recv_sems, ag_send_sems, ag_recv_sems):
        my = lax.axis_index("i")

        barrier_sem = pltpu.get_barrier_semaphore()
        for off in range(1, N_DEV):
            peer = lax.rem(my + off, N_DEV)
            pl.semaphore_signal(
                barrier_sem, inc=1,
                device_id=(peer,), device_id_type=pl.DeviceIdType.MESH,
            )
        pl.semaphore_wait(barrier_sem, N_DEV - 1)

        w1 = w1_ref[...].astype(jnp.bfloat16)
        w2 = w2_ref[...].astype(jnp.bfloat16)

        def partial_chunk(c):
            xb = x_ref[pl.ds(c * CH, CH), :].astype(jnp.bfloat16)
            h = jnp.dot(xb, w1, preferred_element_type=jnp.float32)
            h = jnp.maximum(h, 0.0).astype(jnp.bfloat16)
            return jnp.dot(h, w2, preferred_element_type=jnp.float32)

        rdmas = []
        for k in range(N_DEV - 1):
            c = lax.rem(my + 1 + k, N_DEV)
            rs_send[k] = partial_chunk(c).astype(jnp.bfloat16)
            rdma = pltpu.make_async_remote_copy(
                src_ref=rs_send.at[k],
                dst_ref=rs_recv.at[k],
                send_sem=rs_send_sems.at[k],
                recv_sem=rs_recv_sems.at[k],
                device_id=(c,),
                device_id_type=pl.DeviceIdType.MESH,
            )
            rdma.start()
            rdmas.append(rdma)

        own_ref[...] = partial_chunk(my)
        for k in range(N_DEV - 1):
            rdmas[k].wait_recv()
            own_ref[...] = own_ref[...] + rs_recv[k].astype(jnp.float32)

        out_ref[pl.ds(my * CH, CH), :] = own_ref[...]

        ag_send[...] = own_ref[...].astype(jnp.bfloat16)
        ag_rdmas = []
        for k in range(N_DEV - 1):
            d = lax.rem(my + 1 + k, N_DEV)
            rdma = pltpu.make_async_remote_copy(
                src_ref=ag_send,
                dst_ref=ag_recv.at[k],
                send_sem=ag_send_sems.at[k],
                recv_sem=ag_recv_sems.at[k],
                device_id=(d,),
                device_id_type=pl.DeviceIdType.MESH,
            )
            rdma.start()
            ag_rdmas.append(rdma)

        for k in range(N_DEV - 1):
            ag_rdmas[k].wait_recv()
            c = lax.rem(my - 1 - k + 2 * N_DEV, N_DEV)
            out_ref[pl.ds(c * CH, CH), :] = ag_recv[k].astype(jnp.float32)

        for k in range(N_DEV - 1):
            rdmas[k].wait_send()
            ag_rdmas[k].wait_send()

    n_slots = N_DEV - 1
    return pl.pallas_call(
        body,
        out_shape=jax.ShapeDtypeStruct((M, M), jnp.float32),
        in_specs=[
            pl.BlockSpec(memory_space=pltpu.VMEM),
            pl.BlockSpec(memory_space=pltpu.VMEM),
            pl.BlockSpec(memory_space=pltpu.VMEM),
        ],
        out_specs=pl.BlockSpec(memory_space=pltpu.VMEM),
        scratch_shapes=[
            pltpu.VMEM((CH, M), jnp.float32),
            pltpu.VMEM((n_slots, CH, M), jnp.bfloat16),
            pltpu.VMEM((n_slots, CH, M), jnp.bfloat16),
            pltpu.VMEM((CH, M), jnp.bfloat16),
            pltpu.VMEM((n_slots, CH, M), jnp.bfloat16),
            pltpu.SemaphoreType.DMA((n_slots,)),
            pltpu.SemaphoreType.DMA((n_slots,)),
            pltpu.SemaphoreType.DMA((n_slots,)),
            pltpu.SemaphoreType.DMA((n_slots,)),
        ],
        compiler_params=pltpu.CompilerParams(collective_id=0),
    )(x, W1, W2)
